# baseline (speedup 1.0000x reference)
import jax
import jax.numpy as jnp
from jax.experimental import pallas as pl
from jax.experimental.pallas import tpu as pltpu

H = 16
D = 64
T = 128
NT = 16
S = T * NT
HD = H * D
SCALE = D ** -0.5


def _mm_kernel(a_ref, b_ref, o_ref):
    o_ref[...] = jnp.dot(a_ref[...], b_ref[...],
                         preferred_element_type=jnp.float32)


def _mm(a, b, bn):
    m, k = a.shape
    _, n = b.shape
    return pl.pallas_call(
        _mm_kernel,
        grid=(n // bn,),
        in_specs=[pl.BlockSpec((m, k), lambda t: (0, 0)),
                  pl.BlockSpec((k, bn), lambda t: (0, t))],
        out_specs=pl.BlockSpec((m, bn), lambda t: (0, t)),
        out_shape=jax.ShapeDtypeStruct((m, n), jnp.float32),
    )(a, b)


def _select_kernel(bm_ref, ids_ref, cnt_ref):
    bm = bm_ref[...]
    r = jax.lax.broadcasted_iota(jnp.int32, (H * NT, NT), 0)
    i_row = jax.lax.rem(r, NT)
    j = jax.lax.broadcasted_iota(jnp.int32, (H * NT, NT), 1)
    active = ((j < i_row) & (bm != 0)) | (j == i_row)
    af = active.astype(jnp.float32)
    u = (jax.lax.broadcasted_iota(jnp.int32, (NT, NT), 0)
         <= jax.lax.broadcasted_iota(jnp.int32, (NT, NT), 1)).astype(jnp.float32)
    prefix = jnp.dot(af, u, preferred_element_type=jnp.float32).astype(jnp.int32)
    cnt = prefix[:, NT - 1:NT]
    slot = j
    ids = jnp.zeros((H * NT, NT), jnp.int32)
    for s in range(NT):
        cnt_le = jnp.sum((prefix <= s).astype(jnp.int32), axis=1, keepdims=True)
        ids = ids + jnp.where(slot == s, cnt_le, 0)
    ids = jnp.where(slot >= cnt, i_row, ids)
    ids_ref[...] = ids
    cnt_ref[...] = cnt


def _select(bm2):
    return pl.pallas_call(
        _select_kernel,
        out_shape=(jax.ShapeDtypeStruct((H * NT, NT), jnp.int32),
                   jax.ShapeDtypeStruct((H * NT, 1), jnp.int32)),
    )(bm2)


def _attn_kernel(ids_ref, cnt_ref, q_ref, k_ref, v_ref, o_ref,
                 acc_ref, m_ref, l_ref):
    h = pl.program_id(0)
    i = pl.program_id(1)
    jj = pl.program_id(2)

    @pl.when(jj == 0)
    def _():
        acc_ref[...] = jnp.zeros_like(acc_ref)
        m_ref[...] = jnp.full_like(m_ref, -1e37)
        l_ref[...] = jnp.zeros_like(l_ref)

    cnt = cnt_ref[h * NT + i, 0]

    @pl.when(jj < cnt)
    def _():
        q = q_ref[0]
        k = k_ref[0]
        s = jax.lax.dot_general(q, k, (((1,), (1,)), ((), ())),
                                preferred_element_type=jnp.float32) * SCALE
        jtile = ids_ref[h * NT + i, jj]
        rows = jax.lax.broadcasted_iota(jnp.int32, (T, T), 0)
        cols = jax.lax.broadcasted_iota(jnp.int32, (T, T), 1)
        s = jnp.where((jtile != i) | (rows >= cols), s, -1e30)
        m_prev = m_ref[...]
        m_new = jnp.maximum(m_prev, jnp.max(s, axis=1, keepdims=True))
        alpha = jnp.exp(m_prev - m_new)
        p = jnp.exp(s - m_new)
        l_ref[...] = l_ref[...] * alpha + jnp.sum(p, axis=1, keepdims=True)
        acc_ref[...] = acc_ref[...] * alpha[:, 0:1] + jnp.dot(
            p, v_ref[0], preferred_element_type=jnp.float32)
        m_ref[...] = m_new

    @pl.when(jj == NT - 1)
    def _():
        o_ref[0] = acc_ref[...] / l_ref[:, 0:1]


def _attend(ids, cnt, qkv):
    def qmap(h, i, jj, ids_ref, cnt_ref):
        return (h, i, 0)

    def kmap(h, i, jj, ids_ref, cnt_ref):
        return (H + h, ids_ref[h * NT + i, jj], 0)

    def vmap_(h, i, jj, ids_ref, cnt_ref):
        return (2 * H + h, ids_ref[h * NT + i, jj], 0)

    def omap(h, i, jj, ids_ref, cnt_ref):
        return (h, i, 0)

    grid_spec = pltpu.PrefetchScalarGridSpec(
        num_scalar_prefetch=2,
        grid=(H, NT, NT),
        in_specs=[pl.BlockSpec((1, T, D), qmap),
                  pl.BlockSpec((1, T, D), kmap),
                  pl.BlockSpec((1, T, D), vmap_)],
        out_specs=pl.BlockSpec((1, T, D), omap),
        scratch_shapes=[pltpu.VMEM((T, D), jnp.float32),
                        pltpu.VMEM((T, T), jnp.float32),
                        pltpu.VMEM((T, T), jnp.float32)],
    )
    return pl.pallas_call(
        _attn_kernel,
        grid_spec=grid_spec,
        out_shape=jax.ShapeDtypeStruct((H, S, D), jnp.float32),
    )(ids, cnt, qkv, qkv, qkv)


def _oproj_kernel(a_ref, b_ref, o_ref):
    @pl.when(pl.program_id(0) == 0)
    def _():
        o_ref[...] = jnp.zeros_like(o_ref)

    o_ref[...] += jnp.dot(a_ref[0], b_ref[0],
                          preferred_element_type=jnp.float32)


def _oproj(attn, wo3):
    e = wo3.shape[2]
    return pl.pallas_call(
        _oproj_kernel,
        grid=(H,),
        in_specs=[pl.BlockSpec((1, S, D), lambda h: (h, 0, 0)),
                  pl.BlockSpec((1, D, e), lambda h: (h, 0, 0))],
        out_specs=pl.BlockSpec((S, e), lambda h: (0, 0)),
        out_shape=jax.ShapeDtypeStruct((S, e), jnp.float32),
    )(attn, wo3)


def kernel(x, block_mask, Wq, Wk, Wv, Wo):
    batch, _, e = x.shape
    xf = x.reshape(S, e)
    w = jnp.concatenate([Wq, Wk, Wv], axis=1)
    qkv = _mm(xf, w, 512)
    qkvt = qkv.reshape(S, 3 * H, D).transpose(1, 0, 2)
    bm2 = block_mask.reshape(H * NT, NT).astype(jnp.int32)
    ids, cnt = _select(bm2)
    attn = _attend(ids, cnt, qkvt)
    out = _oproj(attn, Wo.reshape(H, D, -1))
    return out.reshape(batch, S, -1)


# speedup vs baseline: 2.4037x; 2.4037x over previous
import jax
import jax.numpy as jnp
from jax.experimental import pallas as pl
from jax.experimental.pallas import tpu as pltpu

H = 16
D = 64
T = 128
NT = 16
S = T * NT
HD = H * D
SCALE = D ** -0.5


def _mm_kernel(a_ref, b_ref, o_ref):
    o_ref[...] = jnp.dot(a_ref[...], b_ref[...],
                         preferred_element_type=jnp.float32).astype(o_ref.dtype)


def _mm(a, b, bn, out_dtype):
    m, k = a.shape
    _, n = b.shape
    return pl.pallas_call(
        _mm_kernel,
        grid=(n // bn,),
        in_specs=[pl.BlockSpec((m, k), lambda t: (0, 0)),
                  pl.BlockSpec((k, bn), lambda t: (0, t))],
        out_specs=pl.BlockSpec((m, bn), lambda t: (0, t)),
        out_shape=jax.ShapeDtypeStruct((m, n), out_dtype),
    )(a, b)


def _select_kernel(bm_ref, ids_ref, cnt_ref):
    bm = bm_ref[...]
    r = jax.lax.broadcasted_iota(jnp.int32, (H * NT, NT), 0)
    i_row = jax.lax.rem(r, NT)
    j = jax.lax.broadcasted_iota(jnp.int32, (H * NT, NT), 1)
    active = ((j < i_row) & (bm != 0)) | (j == i_row)
    af = active.astype(jnp.float32)
    u = (jax.lax.broadcasted_iota(jnp.int32, (NT, NT), 0)
         <= jax.lax.broadcasted_iota(jnp.int32, (NT, NT), 1)).astype(jnp.float32)
    prefix = jnp.dot(af, u, preferred_element_type=jnp.float32).astype(jnp.int32)
    cnt = prefix[:, NT - 1:NT]
    slot = j
    ids = jnp.zeros((H * NT, NT), jnp.int32)
    for s in range(NT):
        cnt_le = jnp.sum((prefix <= s).astype(jnp.int32), axis=1, keepdims=True)
        ids = ids + jnp.where(slot == s, cnt_le, 0)
    ids = jnp.where(slot >= cnt, i_row, ids)
    ids_ref[...] = ids
    cnt_ref[...] = cnt


def _select(bm2):
    return pl.pallas_call(
        _select_kernel,
        out_shape=(jax.ShapeDtypeStruct((H * NT, NT), jnp.int32),
                   jax.ShapeDtypeStruct((H * NT, 1), jnp.int32)),
    )(bm2)


def _attn_kernel(ids_ref, cnt_ref, q_ref, k_ref, v_ref, o_ref):
    h = pl.program_id(0)
    i = pl.program_id(1)
    cnt = cnt_ref[h * NT + i, 0]
    q = q_ref[0]

    rows = jax.lax.broadcasted_iota(jnp.int32, (T, T), 0)
    cols = jax.lax.broadcasted_iota(jnp.int32, (T, T), 1)

    def tile(jt, m, l, acc, masked):
        k = k_ref[0, pl.ds(jt * T, T), :]
        s = jax.lax.dot_general(q, k, (((1,), (1,)), ((), ())),
                                preferred_element_type=jnp.float32) * SCALE
        if masked:
            s = jnp.where(rows >= cols, s, -1e30)
        m_new = jnp.maximum(m, jnp.max(s, axis=1, keepdims=True))
        alpha = jnp.exp(m - m_new)
        p = jnp.exp(s - m_new)
        l_new = l * alpha + jnp.sum(p, axis=1, keepdims=True)
        v = v_ref[0, pl.ds(jt * T, T), :]
        acc_new = acc * alpha + jax.lax.dot_general(
            p.astype(jnp.bfloat16), v, (((1,), (0,)), ((), ())),
            preferred_element_type=jnp.float32)
        return m_new, l_new, acc_new

    m0 = jnp.full((T, 1), -1e37, jnp.float32)
    l0 = jnp.zeros((T, 1), jnp.float32)
    acc0 = jnp.zeros((T, D), jnp.float32)
    m0, l0, acc0 = tile(i, m0, l0, acc0, masked=True)

    def body(jj, carry):
        m, l, acc = carry
        jt = ids_ref[h * NT + i, jj]
        return tile(jt, m, l, acc, masked=False)

    m, l, acc = jax.lax.fori_loop(0, cnt - 1, body, (m0, l0, acc0))
    o_ref[0] = (acc / l).astype(o_ref.dtype)


def _attend(ids, cnt, qkv):
    return pl.pallas_call(
        _attn_kernel,
        grid_spec=pltpu.PrefetchScalarGridSpec(
            num_scalar_prefetch=2,
            grid=(H, NT),
            in_specs=[pl.BlockSpec((1, T, D), lambda h, i, ids, cnt: (h, i, 0)),
                      pl.BlockSpec((1, S, D),
                                   lambda h, i, ids, cnt: (H + h, 0, 0)),
                      pl.BlockSpec((1, S, D),
                                   lambda h, i, ids, cnt: (2 * H + h, 0, 0))],
            out_specs=pl.BlockSpec((1, T, D), lambda h, i, ids, cnt: (h, i, 0)),
        ),
        out_shape=jax.ShapeDtypeStruct((H, S, D), jnp.bfloat16),
    )(ids, cnt, qkv, qkv, qkv)


def _oproj_kernel(a_ref, b_ref, o_ref):
    @pl.when(pl.program_id(0) == 0)
    def _():
        o_ref[...] = jnp.zeros_like(o_ref)

    o_ref[...] += jnp.dot(a_ref[0], b_ref[0],
                          preferred_element_type=jnp.float32)


def _oproj(attn, wo3):
    e = wo3.shape[2]
    return pl.pallas_call(
        _oproj_kernel,
        grid=(H,),
        in_specs=[pl.BlockSpec((1, S, D), lambda h: (h, 0, 0)),
                  pl.BlockSpec((1, D, e), lambda h: (h, 0, 0))],
        out_specs=pl.BlockSpec((S, e), lambda h: (0, 0)),
        out_shape=jax.ShapeDtypeStruct((S, e), jnp.float32),
    )(attn, wo3)


def kernel(x, block_mask, Wq, Wk, Wv, Wo):
    batch, _, e = x.shape
    xb = x.reshape(S, e).astype(jnp.bfloat16)
    w = jnp.concatenate([Wq, Wk, Wv], axis=1).astype(jnp.bfloat16)
    qkv = _mm(xb, w, 512, jnp.bfloat16)
    qkvt = qkv.reshape(S, 3 * H, D).transpose(1, 0, 2)
    bm2 = block_mask.reshape(H * NT, NT).astype(jnp.int32)
    ids, cnt = _select(bm2)
    attn = _attend(ids, cnt, qkvt)
    out = _oproj(attn, Wo.reshape(H, D, -1).astype(jnp.bfloat16))
    return out.reshape(batch, S, -1)


# speedup vs baseline: 5.5053x; 2.2903x over previous
import jax
import jax.numpy as jnp
from jax.experimental import pallas as pl
from jax.experimental.pallas import tpu as pltpu

H = 16
D = 64
T = 128
NT = 16
S = T * NT
HD = H * D
SCALE = D ** -0.5
QB = 256


def _mm_kernel(a_ref, b_ref, o_ref):
    o_ref[...] = jnp.dot(a_ref[...], b_ref[...],
                         preferred_element_type=jnp.float32).astype(o_ref.dtype)


def _mm(a, b, bn, out_dtype):
    m, k = a.shape
    _, n = b.shape
    return pl.pallas_call(
        _mm_kernel,
        grid=(n // bn,),
        in_specs=[pl.BlockSpec((m, k), lambda t: (0, 0)),
                  pl.BlockSpec((k, bn), lambda t: (0, t))],
        out_specs=pl.BlockSpec((m, bn), lambda t: (0, t)),
        out_shape=jax.ShapeDtypeStruct((m, n), out_dtype),
    )(a, b)


def _select_kernel(bm_ref, am_ref):
    bm = bm_ref[...]
    r = jax.lax.broadcasted_iota(jnp.int32, (H * NT, NT), 0)
    i_row = jax.lax.rem(r, NT)
    j = jax.lax.broadcasted_iota(jnp.int32, (H * NT, NT), 1)
    active = ((j < i_row) & (bm != 0)) | (j == i_row)
    add = jnp.where(active, 0.0, -1e30).astype(jnp.float32)
    g = (jax.lax.broadcasted_iota(jnp.int32, (NT, S), 0)
         == jax.lax.broadcasted_iota(jnp.int32, (NT, S), 1) // T)
    am_ref[...] = jnp.dot(add, g.astype(jnp.float32),
                          preferred_element_type=jnp.float32)


def _select(bm2):
    return pl.pallas_call(
        _select_kernel,
        out_shape=jax.ShapeDtypeStruct((H * NT, S), jnp.float32),
    )(bm2)


def _attn_kernel(q_ref, k_ref, v_ref, am_ref, o_ref):
    i = pl.program_id(1)
    q = q_ref[0]
    k = k_ref[0]
    s = jax.lax.dot_general(q, k, (((1,), (1,)), ((), ())),
                            preferred_element_type=jnp.float32) * SCALE
    am = am_ref[:, 0, :]
    amx = jnp.concatenate(
        [jnp.broadcast_to(am[t:t + 1], (T, S)) for t in range(QB // T)], 0)
    grow = i * QB + jax.lax.broadcasted_iota(jnp.int32, (QB, S), 0)
    gcol = jax.lax.broadcasted_iota(jnp.int32, (QB, S), 1)
    s = jnp.where(gcol <= grow, s + amx, -1e30)
    m = jnp.max(s, axis=1, keepdims=True)
    p = jnp.exp(s - m)
    l = jnp.sum(p, axis=1, keepdims=True)
    o = jnp.dot(p.astype(jnp.bfloat16), v_ref[0],
                preferred_element_type=jnp.float32) / l
    o_ref[0] = o.astype(o_ref.dtype)


def _attend(am, qkv):
    nq = NT * T // QB
    return pl.pallas_call(
        _attn_kernel,
        grid=(H, nq),
        in_specs=[pl.BlockSpec((1, QB, D), lambda h, i: (h, i, 0)),
                  pl.BlockSpec((1, S, D), lambda h, i: (H + h, 0, 0)),
                  pl.BlockSpec((1, S, D), lambda h, i: (2 * H + h, 0, 0)),
                  pl.BlockSpec((QB // T, 1, S),
                               lambda h, i: (h * nq + i, 0, 0))],
        out_specs=pl.BlockSpec((1, QB, D), lambda h, i: (h, i, 0)),
        out_shape=jax.ShapeDtypeStruct((H, S, D), jnp.bfloat16),
    )(qkv, qkv, qkv, am)


def _oproj_kernel(a_ref, b_ref, o_ref):
    @pl.when(pl.program_id(0) == 0)
    def _():
        o_ref[...] = jnp.zeros_like(o_ref)

    o_ref[...] += jnp.dot(a_ref[0], b_ref[0],
                          preferred_element_type=jnp.float32)


def _oproj(attn, wo3):
    e = wo3.shape[2]
    return pl.pallas_call(
        _oproj_kernel,
        grid=(H,),
        in_specs=[pl.BlockSpec((1, S, D), lambda h: (h, 0, 0)),
                  pl.BlockSpec((1, D, e), lambda h: (h, 0, 0))],
        out_specs=pl.BlockSpec((S, e), lambda h: (0, 0)),
        out_shape=jax.ShapeDtypeStruct((S, e), jnp.float32),
    )(attn, wo3)


def kernel(x, block_mask, Wq, Wk, Wv, Wo):
    batch, _, e = x.shape
    xb = x.reshape(S, e).astype(jnp.bfloat16)
    w = jnp.concatenate([Wq, Wk, Wv], axis=1).astype(jnp.bfloat16)
    qkv = _mm(xb, w, 512, jnp.bfloat16)
    qkvt = qkv.reshape(S, 3 * H, D).transpose(1, 0, 2)
    bm2 = block_mask.reshape(H * NT, NT).astype(jnp.int32)
    am = _select(bm2)
    attn = _attend(am.reshape(H * NT, 1, S), qkvt)
    out = _oproj(attn, Wo.reshape(H, D, -1).astype(jnp.bfloat16))
    return out.reshape(batch, S, -1)
